# Initial kernel scaffold; baseline (speedup 1.0000x reference)
import jax
import jax.numpy as jnp
from jax.experimental import pallas as pl


def kernel(node_latent, edge_latent, eW1, eb1, eW2, eb2, nW1, nb1, nW2, nb2, senders, receivers):
    raise NotImplementedError("write your pallas kernel here")


# speedup vs baseline: 3.0186x; 3.0186x over previous
import functools

import jax
import jax.numpy as jnp
from jax import lax
from jax.experimental import pallas as pl
from jax.experimental.pallas import tpu as pltpu
from jax.experimental.pallas import tpu_sc as plsc

N = 10000
E = 320000
D = 128

NC = 2
NS = 16
NW = NC * NS
ET = E // NW
K = 80
NCHUNK = ET // K

_mesh = plsc.VectorSubcoreMesh(core_axis_name="c", subcore_axis_name="s")


@functools.partial(
    pl.kernel,
    out_type=jax.ShapeDtypeStruct((E, D), jnp.float32),
    mesh=_mesh,
    scratch_types=[
        pltpu.VMEM((K,), jnp.int32),
        pltpu.VMEM((K,), jnp.int32),
        pltpu.VMEM((K, D), jnp.float32),
        pltpu.VMEM((K, D), jnp.float32),
        pltpu.SemaphoreType.DMA,
        pltpu.SemaphoreType.DMA,
    ],
)
def _gather_add(ps_hbm, pr_hbm, s_hbm, r_hbm, out_hbm,
                sidx, ridx, buf_s, buf_r, sem_s, sem_r):
    wid = lax.axis_index("s") * NC + lax.axis_index("c")
    tile_base = wid * ET

    def chunk(j, carry):
        base = pl.multiple_of(tile_base + j * K, 8)
        pltpu.sync_copy(s_hbm.at[pl.ds(base, K)], sidx)
        pltpu.sync_copy(r_hbm.at[pl.ds(base, K)], ridx)
        cp_s = pltpu.async_copy(ps_hbm.at[sidx], buf_s, sem_s)
        cp_r = pltpu.async_copy(pr_hbm.at[ridx], buf_r, sem_r)
        cp_s.wait()
        cp_r.wait()

        def row(rr, c2):
            for cc in range(D // 16):
                sl = pl.ds(cc * 16, 16)
                buf_s[rr, sl] = buf_s[rr, sl] + buf_r[rr, sl]
            return c2

        lax.fori_loop(0, K, row, 0)
        pltpu.sync_copy(buf_s, out_hbm.at[pl.ds(base, K)])
        return carry

    lax.fori_loop(0, NCHUNK, chunk, 0)


ZCH = 200
NZ = N // ZCH


@functools.partial(
    pl.kernel,
    out_type=(
        jax.ShapeDtypeStruct((N, D), jnp.float32),
        jax.ShapeDtypeStruct((N, D), jnp.float32),
    ),
    mesh=_mesh,
    scratch_types=[
        pltpu.VMEM_SHARED((N, D), jnp.float32),
        pltpu.VMEM((K,), jnp.int32),
        pltpu.VMEM((K, D), jnp.float32),
        pltpu.VMEM((ZCH, D), jnp.float32),
    ],
)
def _scatter_add(ne_hbm, r_hbm, out0, out1, shared, ridx, rows, zbuf):
    cid = lax.axis_index("c")
    sid = lax.axis_index("s")
    wid = sid * NC + cid
    tile_base = wid * ET

    def zrow(rr, carry):
        for cc in range(D // 16):
            zbuf[rr, pl.ds(cc * 16, 16)] = jnp.zeros((16,), jnp.float32)
        return carry

    lax.fori_loop(0, ZCH, zrow, 0)

    def zcopy(t, carry):
        q = sid + t * NS

        @pl.when(q < NZ)
        def _():
            base = pl.multiple_of(q * ZCH, 8)
            pltpu.sync_copy(zbuf, shared.at[pl.ds(base, ZCH)])

        return carry

    lax.fori_loop(0, (NZ + NS - 1) // NS, zcopy, 0)
    plsc.subcore_barrier()

    def chunk(j, carry):
        base = pl.multiple_of(tile_base + j * K, 8)
        pltpu.sync_copy(r_hbm.at[pl.ds(base, K)], ridx)
        pltpu.sync_copy(ne_hbm.at[pl.ds(base, K)], rows)
        pltpu.sync_copy(rows, shared.at[ridx], add=True)
        return carry

    lax.fori_loop(0, NCHUNK, chunk, 0)
    plsc.subcore_barrier()

    def ocopy(t, carry):
        q = sid + t * NS

        @pl.when((q < NZ) & (cid == 0))
        def _():
            base = pl.multiple_of(q * ZCH, 8)
            pltpu.sync_copy(shared.at[pl.ds(base, ZCH)],
                            out0.at[pl.ds(base, ZCH)])

        @pl.when((q < NZ) & (cid == 1))
        def _():
            base = pl.multiple_of(q * ZCH, 8)
            pltpu.sync_copy(shared.at[pl.ds(base, ZCH)],
                            out1.at[pl.ds(base, ZCH)])

        return carry

    lax.fori_loop(0, (NZ + NS - 1) // NS, ocopy, 0)


def _pre_body(nl_ref, ws_ref, wr_ref, ps_ref, pr_ref):
    x = nl_ref[...]
    ps_ref[...] = jnp.dot(x, ws_ref[...], preferred_element_type=jnp.float32)
    pr_ref[...] = jnp.dot(x, wr_ref[...], preferred_element_type=jnp.float32)


def _precompute(node_latent, w1s, w1r):
    blk = 2000
    grid = N // blk
    return pl.pallas_call(
        _pre_body,
        grid=(grid,),
        in_specs=[
            pl.BlockSpec((blk, D), lambda i: (i, 0)),
            pl.BlockSpec((D, D), lambda i: (0, 0)),
            pl.BlockSpec((D, D), lambda i: (0, 0)),
        ],
        out_specs=[
            pl.BlockSpec((blk, D), lambda i: (i, 0)),
            pl.BlockSpec((blk, D), lambda i: (i, 0)),
        ],
        out_shape=[
            jax.ShapeDtypeStruct((N, D), jnp.float32),
            jax.ShapeDtypeStruct((N, D), jnp.float32),
        ],
    )(node_latent, w1s, w1r)


def _edge_body(el_ref, g_ref, w1_ref, b1_ref, w2_ref, b2_ref, out_ref):
    el = el_ref[...]
    h = jnp.dot(el, w1_ref[...], preferred_element_type=jnp.float32)
    h = jnp.maximum(h + g_ref[...] + b1_ref[...], 0.0)
    out_ref[...] = el + jnp.dot(
        h, w2_ref[...], preferred_element_type=jnp.float32) + b2_ref[...]


def _edge_mlp(edge_latent, g, w1e, b1, w2, b2):
    blk = 2560
    grid = E // blk
    return pl.pallas_call(
        _edge_body,
        grid=(grid,),
        in_specs=[
            pl.BlockSpec((blk, D), lambda i: (i, 0)),
            pl.BlockSpec((blk, D), lambda i: (i, 0)),
            pl.BlockSpec((D, D), lambda i: (0, 0)),
            pl.BlockSpec((1, D), lambda i: (0, 0)),
            pl.BlockSpec((D, D), lambda i: (0, 0)),
            pl.BlockSpec((1, D), lambda i: (0, 0)),
        ],
        out_specs=pl.BlockSpec((blk, D), lambda i: (i, 0)),
        out_shape=jax.ShapeDtypeStruct((E, D), jnp.float32),
    )(edge_latent, g, w1e, b1, w2, b2)


def _node_body(nl_ref, a0_ref, a1_ref, w1n_ref, w1a_ref, b1_ref, w2_ref,
               b2_ref, out_ref):
    x = nl_ref[...]
    agg = a0_ref[...] + a1_ref[...]
    h = jnp.dot(x, w1n_ref[...], preferred_element_type=jnp.float32)
    h = h + jnp.dot(agg, w1a_ref[...], preferred_element_type=jnp.float32)
    h = jnp.maximum(h + b1_ref[...], 0.0)
    out_ref[...] = x + jnp.dot(
        h, w2_ref[...], preferred_element_type=jnp.float32) + b2_ref[...]


def _node_mlp(node_latent, a0, a1, w1n, w1a, b1, w2, b2):
    blk = 2000
    grid = N // blk
    return pl.pallas_call(
        _node_body,
        grid=(grid,),
        in_specs=[
            pl.BlockSpec((blk, D), lambda i: (i, 0)),
            pl.BlockSpec((blk, D), lambda i: (i, 0)),
            pl.BlockSpec((blk, D), lambda i: (i, 0)),
            pl.BlockSpec((D, D), lambda i: (0, 0)),
            pl.BlockSpec((D, D), lambda i: (0, 0)),
            pl.BlockSpec((1, D), lambda i: (0, 0)),
            pl.BlockSpec((D, D), lambda i: (0, 0)),
            pl.BlockSpec((1, D), lambda i: (0, 0)),
        ],
        out_specs=pl.BlockSpec((blk, D), lambda i: (i, 0)),
        out_shape=jax.ShapeDtypeStruct((N, D), jnp.float32),
    )(node_latent, a0, a1, w1n, w1a, b1, w2, b2)


def kernel(node_latent, edge_latent, eW1, eb1, eW2, eb2,
           nW1, nb1, nW2, nb2, senders, receivers):
    w1e, w1s, w1r = eW1[:D], eW1[D:2 * D], eW1[2 * D:]
    ps, pr = _precompute(node_latent, w1s, w1r)
    g = _gather_add(ps, pr, senders, receivers)
    new_edge = _edge_mlp(edge_latent, g, w1e, eb1.reshape(1, D), eW2,
                         eb2.reshape(1, D))
    a0, a1 = _scatter_add(new_edge, receivers)
    new_node = _node_mlp(node_latent, a0, a1, nW1[:D], nW1[D:],
                         nb1.reshape(1, D), nW2, nb2.reshape(1, D))
    return new_node, new_edge
